# baseline (speedup 1.0000x reference)
import functools

import jax
import jax.numpy as jnp
from jax import lax
from jax.experimental import pallas as pl
from jax.experimental.pallas import tpu as pltpu
from jax.experimental.pallas import tpu_sc as plsc

VOCAB = 100000
EMBED_DIM = 64
PAD_DIM = 128
B = 16384
L = 50
TOT = B * L

_info = plsc.get_sparse_core_info()
NC, NS = _info.num_cores, _info.num_subcores

BLK = 128
NBLK = B // (NC * BLK)
BAT_TEC = BLK // NS
SUB = 4
ROWS = SUB * L

_mesh = plsc.VectorSubcoreMesh(core_axis_name="c", subcore_axis_name="s")


@functools.partial(
    pl.kernel,
    mesh=_mesh,
    out_type=jax.ShapeDtypeStruct((L, EMBED_DIM, B), jnp.float32),
    scratch_types=[
        pltpu.VMEM((ROWS,), jnp.int32),
        pltpu.VMEM((ROWS,), jnp.int32),
        pltpu.VMEM((ROWS, EMBED_DIM), jnp.float32),
        pltpu.VMEM((ROWS, EMBED_DIM), jnp.float32),
        pltpu.VMEM((L, EMBED_DIM, BAT_TEC), jnp.float32),
        pltpu.VMEM_SHARED((L, EMBED_DIM, BLK), jnp.float32),
        pltpu.VMEM_SHARED((L, EMBED_DIM, BLK), jnp.float32),
        pltpu.SemaphoreType.DMA,
        pltpu.SemaphoreType.DMA,
        pltpu.SemaphoreType.DMA,
        pltpu.SemaphoreType.DMA,
        pltpu.SemaphoreType.DMA,
        pltpu.SemaphoreType.DMA,
    ],
    compiler_params=pltpu.CompilerParams(
        use_tc_tiling_on_sc=False, needs_layout_passes=False
    ),
)
def _gather_sc(
    idx_hbm, table_hbm, out_hbm,
    idx0, idx1, rows0, rows1, trans, stage0, stage1,
    is0, is1, gs0, gs1, ss0, ss1,
):
    c = lax.axis_index("c")
    s = lax.axis_index("s")
    idxb = (idx0, idx1)
    rowsb = (rows0, rows1)
    stageb = (stage0, stage1)
    isem = (is0, is1)
    gsem = (gs0, gs1)
    ssem = (ss0, ss1)

    def sub_off(blk, u2):
        bat = c * (B // NC) + blk * BLK + s * BAT_TEC + u2 * SUB
        return pl.multiple_of(bat * L, ROWS)

    def i_start(blk, u2):
        pltpu.async_copy(
            idx_hbm.at[pl.ds(sub_off(blk, u2), ROWS)], idxb[u2], isem[u2]
        )

    def i_wait(u2):
        pltpu.make_async_copy(
            idx_hbm.at[pl.ds(0, ROWS)], idxb[u2], isem[u2]
        ).wait()

    def g_start(u2):
        pltpu.async_copy(table_hbm.at[idxb[u2]], rowsb[u2], gsem[u2])

    def g_wait(u2):
        pltpu.make_async_copy(
            table_hbm.at[idxb[u2]], rowsb[u2], gsem[u2]
        ).wait()

    def transpose(u2):
        iota = lax.iota(jnp.int32, 16)

        @pl.loop(0, L)
        def _(l):
            for j in range(SUB):
                for d0 in range(0, EMBED_DIM, 16):
                    vec = rowsb[u2][j * L + l, pl.ds(d0, 16)]
                    plsc.store_scatter(
                        trans,
                        [
                            jnp.full((16,), l, jnp.int32),
                            d0 + iota,
                            jnp.full((16,), u2 * SUB + j, jnp.int32),
                        ],
                        vec,
                    )

    def l_copy(p):
        pltpu.sync_copy(trans, stageb[p].at[:, :, pl.ds(s * BAT_TEC, BAT_TEC)])

    def st_start(blk, p):
        col = pl.multiple_of(c * (B // NC) + blk * BLK, BLK)
        pltpu.async_copy(stageb[p], out_hbm.at[:, :, pl.ds(col, BLK)], ssem[p])

    def st_wait(p):
        pltpu.make_async_copy(
            stageb[p], out_hbm.at[:, :, pl.ds(0, BLK)], ssem[p]
        ).wait()

    @pl.loop(0, NBLK, step=2)
    def _(blk0):
        for bp in range(2):
            blk = blk0 + bp
            for u2 in range(2):
                i_start(blk, u2)
                i_wait(u2)
                g_start(u2)
                g_wait(u2)
                transpose(u2)
            plsc.subcore_barrier()
            l_copy(bp)
            plsc.subcore_barrier()
            @pl.when(s == 0)
            def _():
                st_start(blk, bp)
                st_wait(bp)
            plsc.subcore_barrier()


def kernel(indices, table):
    flat = indices.reshape(-1).astype(jnp.int32)
    out = _gather_sc(flat, table)
    return jnp.transpose(out, (2, 0, 1))


# speedup vs baseline: 1.5006x; 1.5006x over previous
import functools

import jax
import jax.numpy as jnp
from jax import lax
from jax.experimental import pallas as pl
from jax.experimental.pallas import tpu as pltpu
from jax.experimental.pallas import tpu_sc as plsc

VOCAB = 100000
EMBED_DIM = 64
PAD_DIM = 128
B = 16384
L = 50
TOT = B * L

_info = plsc.get_sparse_core_info()
NC, NS = _info.num_cores, _info.num_subcores

BLK = 128
NBLK = B // (NC * BLK)
BAT_TEC = BLK // NS
SUB = 4
ROWS = SUB * L

_mesh = plsc.VectorSubcoreMesh(core_axis_name="c", subcore_axis_name="s")


@functools.partial(
    pl.kernel,
    mesh=_mesh,
    out_type=jax.ShapeDtypeStruct((L, EMBED_DIM, B), jnp.float32),
    scratch_types=[
        pltpu.VMEM((ROWS,), jnp.int32),
        pltpu.VMEM((ROWS,), jnp.int32),
        pltpu.VMEM((ROWS, EMBED_DIM), jnp.float32),
        pltpu.VMEM((ROWS, EMBED_DIM), jnp.float32),
        pltpu.VMEM((L, EMBED_DIM, BAT_TEC), jnp.float32),
        pltpu.VMEM_SHARED((L, EMBED_DIM, BLK), jnp.float32),
        pltpu.VMEM_SHARED((L, EMBED_DIM, BLK), jnp.float32),
        pltpu.SemaphoreType.DMA,
        pltpu.SemaphoreType.DMA,
        pltpu.SemaphoreType.DMA,
        pltpu.SemaphoreType.DMA,
        pltpu.SemaphoreType.DMA,
        pltpu.SemaphoreType.DMA,
    ],
    compiler_params=pltpu.CompilerParams(
        use_tc_tiling_on_sc=False, needs_layout_passes=False
    ),
)
def _gather_sc(
    idx_hbm, table_hbm, out_hbm,
    idx0, idx1, rows0, rows1, trans, stage0, stage1,
    is0, is1, gs0, gs1, ss0, ss1,
):
    c = lax.axis_index("c")
    s = lax.axis_index("s")
    idxb = (idx0, idx1)
    rowsb = (rows0, rows1)
    stageb = (stage0, stage1)
    isem = (is0, is1)
    gsem = (gs0, gs1)
    ssem = (ss0, ss1)

    def sub_off(blk, u2):
        bat = c * (B // NC) + blk * BLK + s * BAT_TEC + u2 * SUB
        return pl.multiple_of(bat * L, ROWS)

    def i_start(blk, u2):
        pltpu.async_copy(
            idx_hbm.at[pl.ds(sub_off(blk, u2), ROWS)], idxb[u2], isem[u2]
        )

    def i_wait(u2):
        pltpu.make_async_copy(
            idx_hbm.at[pl.ds(0, ROWS)], idxb[u2], isem[u2]
        ).wait()

    def g_start(u2):
        pltpu.async_copy(table_hbm.at[idxb[u2]], rowsb[u2], gsem[u2])

    def g_wait(u2):
        pltpu.make_async_copy(
            table_hbm.at[idxb[u2]], rowsb[u2], gsem[u2]
        ).wait()

    def transpose(u2):
        iota = lax.iota(jnp.int32, 16)

        @pl.loop(0, L)
        def _(l):
            for j in range(SUB):
                for d0 in range(0, EMBED_DIM, 16):
                    vec = rowsb[u2][j * L + l, pl.ds(d0, 16)]
                    plsc.store_scatter(
                        trans,
                        [
                            jnp.full((16,), l, jnp.int32),
                            d0 + iota,
                            jnp.full((16,), u2 * SUB + j, jnp.int32),
                        ],
                        vec,
                    )

    def l_copy(p):
        pltpu.sync_copy(trans, stageb[p].at[:, :, pl.ds(s * BAT_TEC, BAT_TEC)])

    def st_start(blk, p):
        col = pl.multiple_of(c * (B // NC) + blk * BLK, BLK)
        pltpu.async_copy(stageb[p], out_hbm.at[:, :, pl.ds(col, BLK)], ssem[p])

    def st_wait(p):
        pltpu.make_async_copy(
            stageb[p], out_hbm.at[:, :, pl.ds(0, BLK)], ssem[p]
        ).wait()

    i_start(0, 0)
    i_start(0, 1)
    i_wait(0)
    g_start(0)

    @pl.loop(0, NBLK, step=2)
    def _(blk0):
        for bp in range(2):
            blk = blk0 + bp
            for u2 in range(2):
                g_wait(u2)
                @pl.when(blk + 1 < NBLK)
                def _():
                    i_start(blk + 1, u2)
                if u2 == 0:
                    i_wait(1)
                    g_start(1)
                else:
                    @pl.when(blk + 1 < NBLK)
                    def _():
                        i_wait(0)
                        g_start(0)
                transpose(u2)
            @pl.when(jnp.logical_and(s == 0, blk >= 2))
            def _():
                st_wait(bp)
            plsc.subcore_barrier()
            l_copy(bp)
            plsc.subcore_barrier()
            @pl.when(s == 0)
            def _():
                st_start(blk, bp)

    @pl.when(s == 0)
    def _():
        st_wait(0)
        st_wait(1)
    plsc.subcore_barrier()


def kernel(indices, table):
    flat = indices.reshape(-1).astype(jnp.int32)
    out = _gather_sc(flat, table)
    return jnp.transpose(out, (2, 0, 1))


# speedup vs baseline: 2.0069x; 1.3374x over previous
import functools

import jax
import jax.numpy as jnp
from jax import lax
from jax.experimental import pallas as pl
from jax.experimental.pallas import tpu as pltpu
from jax.experimental.pallas import tpu_sc as plsc

VOCAB = 100000
EMBED_DIM = 64
B = 16384
L = 50
TOT = B * L

_info = plsc.get_sparse_core_info()
NC, NS = _info.num_cores, _info.num_subcores

BLK = 128
NBLK = B // (NC * BLK)
BAT_TEC = BLK // NS
ROWS = BAT_TEC * L

_mesh = plsc.VectorSubcoreMesh(core_axis_name="c", subcore_axis_name="s")


@functools.partial(
    pl.kernel,
    mesh=_mesh,
    out_type=jax.ShapeDtypeStruct((L, EMBED_DIM, B), jnp.float32),
    scratch_types=[
        pltpu.VMEM((ROWS,), jnp.int32),
        pltpu.VMEM((ROWS,), jnp.int32),
        pltpu.VMEM((ROWS, EMBED_DIM), jnp.float32),
        pltpu.VMEM((ROWS, EMBED_DIM), jnp.float32),
        pltpu.VMEM((L, EMBED_DIM, BAT_TEC), jnp.float32),
        pltpu.VMEM_SHARED((L, EMBED_DIM, BLK), jnp.float32),
        pltpu.VMEM_SHARED((L, EMBED_DIM, BLK), jnp.float32),
        pltpu.SemaphoreType.DMA,
        pltpu.SemaphoreType.DMA,
        pltpu.SemaphoreType.DMA,
        pltpu.SemaphoreType.DMA,
        pltpu.SemaphoreType.DMA,
        pltpu.SemaphoreType.DMA,
    ],
    compiler_params=pltpu.CompilerParams(
        use_tc_tiling_on_sc=False, needs_layout_passes=False
    ),
)
def _gather_sc(
    idx_hbm, table_hbm, out_hbm,
    idx0, idx1, rows0, rows1, trans, stage0, stage1,
    is0, is1, gs0, gs1, ss0, ss1,
):
    c = lax.axis_index("c")
    s = lax.axis_index("s")
    idxb = (idx0, idx1)
    rowsb = (rows0, rows1)
    stageb = (stage0, stage1)
    isem = (is0, is1)
    gsem = (gs0, gs1)
    ssem = (ss0, ss1)

    def blk_off(blk):
        bat = c * (B // NC) + blk * BLK + s * BAT_TEC
        return pl.multiple_of(bat * L, ROWS)

    def i_start(blk, p):
        pltpu.async_copy(
            idx_hbm.at[pl.ds(blk_off(blk), ROWS)], idxb[p], isem[p]
        )

    def i_wait(p):
        pltpu.make_async_copy(
            idx_hbm.at[pl.ds(0, ROWS)], idxb[p], isem[p]
        ).wait()

    def g_start(p):
        pltpu.async_copy(table_hbm.at[idxb[p]], rowsb[p], gsem[p])

    def g_wait(p):
        pltpu.make_async_copy(
            table_hbm.at[idxb[p]], rowsb[p], gsem[p]
        ).wait()

    iota = lax.iota(jnp.int32, 16)
    dvecs = [d0 + iota for d0 in range(0, EMBED_DIM, 16)]
    jvecs = [jnp.full((16,), j, jnp.int32) for j in range(BAT_TEC)]

    combos = [
        (j, di, d0)
        for j in range(BAT_TEC)
        for di, d0 in enumerate(range(0, EMBED_DIM, 16))
    ]

    def transpose(p):
        @plsc.parallel_loop(0, L)
        def _(l):
            lvec = jnp.full((16,), l, jnp.int32)
            for g in range(0, len(combos), 8):
                grp = combos[g:g + 8]
                vecs = [
                    rowsb[p][j * L + l, pl.ds(d0, 16)] for j, _, d0 in grp
                ]
                for (j, di, _), vec in zip(grp, vecs):
                    plsc.store_scatter(
                        trans, [lvec, dvecs[di], jvecs[j]], vec
                    )

    def l_copy(p):
        pltpu.sync_copy(trans, stageb[p].at[:, :, pl.ds(s * BAT_TEC, BAT_TEC)])

    def st_start(blk, p):
        col = pl.multiple_of(c * (B // NC) + blk * BLK, BLK)
        pltpu.async_copy(stageb[p], out_hbm.at[:, :, pl.ds(col, BLK)], ssem[p])

    def st_wait(p):
        pltpu.make_async_copy(
            stageb[p], out_hbm.at[:, :, pl.ds(0, BLK)], ssem[p]
        ).wait()

    i_start(0, 0)
    i_start(1, 1)
    i_wait(0)
    g_start(0)

    @pl.loop(0, NBLK, step=2)
    def _(blk0):
        for bp in range(2):
            blk = blk0 + bp
            nbp = 1 - bp
            g_wait(bp)
            @pl.when(blk + 2 < NBLK)
            def _():
                i_start(blk + 2, bp)
            @pl.when(blk + 1 < NBLK)
            def _():
                i_wait(nbp)
                g_start(nbp)
            transpose(bp)
            @pl.when(jnp.logical_and(s == 0, blk >= 2))
            def _():
                st_wait(bp)
            plsc.subcore_barrier()
            l_copy(bp)
            plsc.subcore_barrier()
            @pl.when(s == 0)
            def _():
                st_start(blk, bp)

    @pl.when(s == 0)
    def _():
        st_wait(0)
        st_wait(1)
    plsc.subcore_barrier()


def kernel(indices, table):
    flat = indices.reshape(-1).astype(jnp.int32)
    out = _gather_sc(flat, table)
    return jnp.transpose(out, (2, 0, 1))
